# baseline (speedup 1.0000x reference)
import functools

import jax
import jax.numpy as jnp
from jax import lax
from jax.experimental import pallas as pl
from jax.experimental.pallas import tpu as pltpu
from jax.experimental.pallas import tpu_sc as plsc

B = 1024
D = 64
TOKEN_NUM = 1000000
X_MAX = 100.0
ALPHA = 0.75

NC = 2
NS = 16
NW = NC * NS
BPW = B // NW
NT = TOKEN_NUM // 8


def _sc_gather(ii_hbm, ij_hbm, emb_i3, emb_j3, bi3, bj3,
               wti_out, wtj_out, bti_out, btj_out,
               ii_v, ij_v, sem):
    wid = lax.axis_index("s") * NC + lax.axis_index("c")
    base = wid * BPW
    chunk = pl.multiple_of((base // 128) * 128, 128)
    off = base - chunk
    pltpu.sync_copy(ii_hbm.at[pl.ds(chunk, 128)], ii_v)
    pltpu.sync_copy(ij_hbm.at[pl.ds(chunk, 128)], ij_v)
    copies = []
    for g in range(BPW // 16):
        vec_i = ii_v[pl.ds(off + g * 16, 16)]
        vec_j = ij_v[pl.ds(off + g * 16, 16)]
        for l in range(16):
            k_i = lax.shift_right_logical(vec_i[l], 3)
            k_j = lax.shift_right_logical(vec_j[l], 3)
            row = g * 16 + l
            dst = pl.ds(base + row, 1)
            copies.append(pltpu.async_copy(
                emb_i3.at[pl.ds(k_i, 1)], wti_out.at[dst], sem))
            copies.append(pltpu.async_copy(
                emb_j3.at[pl.ds(k_j, 1)], wtj_out.at[dst], sem))
            copies.append(pltpu.async_copy(
                bi3.at[pl.ds(k_i, 1)], bti_out.at[dst], sem))
            copies.append(pltpu.async_copy(
                bj3.at[pl.ds(k_j, 1)], btj_out.at[dst], sem))
    for cp in copies:
        cp.wait()


_sc_kernel = functools.partial(
    pl.kernel,
    out_type=(
        jax.ShapeDtypeStruct((B, 8, D), jnp.float32),
        jax.ShapeDtypeStruct((B, 8, D), jnp.float32),
        jax.ShapeDtypeStruct((B, 8, 1), jnp.float32),
        jax.ShapeDtypeStruct((B, 8, 1), jnp.float32),
    ),
    mesh=plsc.VectorSubcoreMesh(core_axis_name="c", subcore_axis_name="s"),
    scratch_types=[
        pltpu.VMEM((128,), jnp.int32),
        pltpu.VMEM((128,), jnp.int32),
        pltpu.SemaphoreType.DMA,
    ],
)(_sc_gather)


ROW_BLK = 128


def _tc_outer(xij_ref, subi_ref, subj_ref, wti_ref, wtj_ref,
              bti_ref, btj_ref, out_ref):
    xf = xij_ref[...].astype(jnp.float32)
    t = jnp.log(xf)
    fx = jnp.where(xf >= X_MAX, jnp.float32(1.0),
                   jnp.exp(ALPHA * jnp.log(xf * (1.0 / X_MAX))))

    sub_i = subi_ref[...]
    sub_j = subj_ref[...]
    io8 = lax.broadcasted_iota(jnp.int32, (ROW_BLK, 8), 1)
    oh_i = (io8 == sub_i).astype(jnp.float32)
    oh_j = (io8 == sub_j).astype(jnp.float32)

    wi = jnp.sum(wti_ref[...] * oh_i[:, :, None], axis=1)
    wj = jnp.sum(wtj_ref[...] * oh_j[:, :, None], axis=1)
    dots = jnp.sum(wi * wj, axis=1, keepdims=True)
    b_i = jnp.sum(bti_ref[...] * oh_i, axis=1, keepdims=True)
    b_j = jnp.sum(btj_ref[...] * oh_j, axis=1, keepdims=True)

    s = dots + b_i + b_j
    diff = s - t
    out_ref[...] = fx * diff * diff


def kernel(x, emb_i, emb_j, bi, bj):
    idx_i = x[:, 0]
    idx_j = x[:, 1]
    xij2 = x[:, 2].reshape(1, B)
    sub_i = (idx_i & 7).reshape(B, 1)
    sub_j = (idx_j & 7).reshape(B, 1)

    wti, wtj, bti, btj = _sc_kernel(
        idx_i, idx_j,
        emb_i.reshape(NT, 8, D), emb_j.reshape(NT, 8, D),
        bi.reshape(NT, 8, 1), bj.reshape(NT, 8, 1))

    out2 = pl.pallas_call(
        _tc_outer,
        grid=(B // ROW_BLK,),
        in_specs=[
            pl.BlockSpec((1, B), lambda i: (0, 0)),
            pl.BlockSpec((ROW_BLK, 1), lambda i: (i, 0)),
            pl.BlockSpec((ROW_BLK, 1), lambda i: (i, 0)),
            pl.BlockSpec((ROW_BLK, 8, D), lambda i: (i, 0, 0)),
            pl.BlockSpec((ROW_BLK, 8, D), lambda i: (i, 0, 0)),
            pl.BlockSpec((ROW_BLK, 8), lambda i: (i, 0)),
            pl.BlockSpec((ROW_BLK, 8), lambda i: (i, 0)),
        ],
        out_specs=pl.BlockSpec((ROW_BLK, B), lambda i: (i, 0)),
        out_shape=jax.ShapeDtypeStruct((B, B), jnp.float32),
    )(xij2, sub_i, sub_j, wti, wtj,
      bti.reshape(B, 8), btj.reshape(B, 8))

    return out2.reshape(B, 1, B)


# speedup vs baseline: 2.4005x; 2.4005x over previous
import functools

import jax
import jax.numpy as jnp
from jax import lax
from jax.experimental import pallas as pl
from jax.experimental.pallas import tpu as pltpu
from jax.experimental.pallas import tpu_sc as plsc

B = 1024
D = 64
TOKEN_NUM = 1000000
X_MAX = 100.0
ALPHA = 0.75

NC = 2
NS = 16
NW = NC * NS
BPW = B // NW
NT = TOKEN_NUM // 8
PK = 128


def _sc_gather(ii_hbm, ij_hbm, emb_i3, emb_j3,
               packed_out,
               ii_v, ij_v, ri_v, rj_v, p_v, sem):
    wid = lax.axis_index("s") * NC + lax.axis_index("c")
    base = wid * BPW
    chunk = pl.multiple_of((base // 128) * 128, 128)
    off = base - chunk
    pltpu.sync_copy(ii_hbm.at[pl.ds(chunk, 128)], ii_v)
    pltpu.sync_copy(ij_hbm.at[pl.ds(chunk, 128)], ij_v)

    copies = []
    subs = []
    for g in range(BPW // 16):
        vec_i = ii_v[pl.ds(off + g * 16, 16)]
        vec_j = ij_v[pl.ds(off + g * 16, 16)]
        for l in range(16):
            r_i = vec_i[l]
            r_j = vec_j[l]
            k_i = lax.shift_right_logical(r_i, 3)
            k_j = lax.shift_right_logical(r_j, 3)
            row = g * 16 + l
            subs.append((lax.rem(r_i, 8), lax.rem(r_j, 8)))
            copies.append(pltpu.async_copy(
                emb_i3.at[pl.ds(k_i, 1)], ri_v.at[pl.ds(row, 1)], sem))
            copies.append(pltpu.async_copy(
                emb_j3.at[pl.ds(k_j, 1)], rj_v.at[pl.ds(row, 1)], sem))
    for cp in copies:
        cp.wait()

    for row in range(BPW):
        s_i, s_j = subs[row]
        for c in range(D // 16):
            p_v[row, pl.ds(c * 16, 16)] = (
                ri_v[row, s_i, pl.ds(c * 16, 16)] *
                rj_v[row, s_j, pl.ds(c * 16, 16)])

    pltpu.sync_copy(p_v, packed_out.at[pl.ds(base, BPW)])


_sc_kernel = functools.partial(
    pl.kernel,
    out_type=jax.ShapeDtypeStruct((B, PK), jnp.float32),
    mesh=plsc.VectorSubcoreMesh(core_axis_name="c", subcore_axis_name="s"),
    scratch_types=[
        pltpu.VMEM((128,), jnp.int32),
        pltpu.VMEM((128,), jnp.int32),
        pltpu.VMEM((BPW, 8, D), jnp.float32),
        pltpu.VMEM((BPW, 8, D), jnp.float32),
        pltpu.VMEM((BPW, PK), jnp.float32),
        pltpu.SemaphoreType.DMA,
    ],
)(_sc_gather)


def _tc_bias(idx_ref, bi_hbm, bj_hbm, out_ref, bi_s, bj_s, sem):
    def chunk(c, carry):
        cbase = c * 16
        for l in range(16):
            j = cbase + l
            r_i = idx_ref[0, j]
            r_j = idx_ref[1, j]
            pltpu.make_async_copy(
                bi_hbm.at[pl.ds(r_i, 1), :], bi_s.at[pl.ds(j, 1), :],
                sem).start()
            pltpu.make_async_copy(
                bj_hbm.at[pl.ds(r_j, 1), :], bj_s.at[pl.ds(j, 1), :],
                sem).start()
        return carry

    lax.fori_loop(0, B // 16, chunk, 0)
    pltpu.make_async_copy(bi_hbm.at[pl.ds(0, B), :], bi_s, sem).wait()
    pltpu.make_async_copy(bj_hbm.at[pl.ds(0, B), :], bj_s, sem).wait()
    out_ref[...] = bi_s[...] + bj_s[...]


def _bias_gather(idx2, bi, bj):
    return pl.pallas_call(
        _tc_bias,
        in_specs=[
            pl.BlockSpec(memory_space=pltpu.SMEM),
            pl.BlockSpec(memory_space=pl.ANY),
            pl.BlockSpec(memory_space=pl.ANY),
        ],
        out_shape=jax.ShapeDtypeStruct((B, 1), jnp.float32),
        scratch_shapes=[
            pltpu.VMEM((B, 1), jnp.float32),
            pltpu.VMEM((B, 1), jnp.float32),
            pltpu.SemaphoreType.DMA,
        ],
    )(idx2, bi, bj)


ROW_BLK = 128


def _tc_outer(xij_ref, packed_ref, bsum_ref, out_ref):
    xf = xij_ref[...].astype(jnp.float32)
    t = jnp.log(xf)
    fx = jnp.where(xf >= X_MAX, jnp.float32(1.0),
                   jnp.exp(ALPHA * jnp.log(xf * (1.0 / X_MAX))))
    dots = jnp.sum(packed_ref[:, :D], axis=1, keepdims=True)
    s = dots + bsum_ref[...]
    diff = s - t
    out_ref[...] = fx * diff * diff


def kernel(x, emb_i, emb_j, bi, bj):
    idx_i = x[:, 0]
    idx_j = x[:, 1]
    xij2 = x[:, 2].reshape(1, B)
    idx2 = jnp.stack([idx_i, idx_j])

    packed = _sc_kernel(
        idx_i, idx_j, emb_i.reshape(NT, 8, D), emb_j.reshape(NT, 8, D))
    bsum = _bias_gather(idx2, bi, bj)

    out2 = pl.pallas_call(
        _tc_outer,
        grid=(B // ROW_BLK,),
        in_specs=[
            pl.BlockSpec((1, B), lambda i: (0, 0)),
            pl.BlockSpec((ROW_BLK, PK), lambda i: (i, 0)),
            pl.BlockSpec((ROW_BLK, 1), lambda i: (i, 0)),
        ],
        out_specs=pl.BlockSpec((ROW_BLK, B), lambda i: (i, 0)),
        out_shape=jax.ShapeDtypeStruct((B, B), jnp.float32),
    )(xij2, packed, bsum)

    return out2.reshape(B, 1, B)
